# baseline (speedup 1.0000x reference)
import functools

import jax
import jax.numpy as jnp
from jax import lax
from jax.experimental import pallas as pl
from jax.experimental.pallas import tpu as pltpu
from jax.experimental.pallas import tpu_sc as plsc

_SIZES = (6, 10, 2, 1, 1, 1, 1, 2, 1, 1, 2, 2)
_OFF = (0, 6, 16, 18, 19, 20, 21, 22, 24, 25, 26, 28)
_NF = 12
_B = 16384
_NC, _NS, _L = 2, 16, 16
_NW = _NC * _NS
_ROWS = _B // _NW
_CHUNK = _ROWS * _NF
_GROUPS = _ROWS // _L

_mesh = plsc.VectorSubcoreMesh(core_axis_name="c", subcore_axis_name="s")


@functools.partial(
    pl.kernel,
    out_type=jax.ShapeDtypeStruct((_B,), jnp.float32),
    mesh=_mesh,
    scratch_types=[
        pltpu.VMEM((_CHUNK,), jnp.int32),
        pltpu.VMEM((_CHUNK,), jnp.float32),
        pltpu.VMEM((32,), jnp.float32),
        pltpu.VMEM((_L,), jnp.float32),
        pltpu.VMEM((_ROWS,), jnp.float32),
    ],
    compiler_params=pltpu.CompilerParams(needs_layout_passes=False),
)
def _sc_fm(x_hbm, w_hbm, tab_hbm, bias_hbm, out_hbm, x_v, w_v, tab_v, bias_v, out_v):
    wid = lax.axis_index("s") * _NC + lax.axis_index("c")
    base = wid * _CHUNK
    pltpu.sync_copy(x_hbm.at[pl.ds(base, _CHUNK)], x_v)
    pltpu.sync_copy(w_hbm.at[pl.ds(base, _CHUNK)], w_v)
    pltpu.sync_copy(tab_hbm, tab_v)
    pltpu.sync_copy(bias_hbm, bias_v)
    iota = lax.iota(jnp.int32, _L) * _NF
    for g in range(_GROUPS):
        acc = bias_v[...]
        for i in range(_NF):
            ids = iota + (g * _L * _NF + i)
            xv = plsc.load_gather(x_v, [ids])
            tv = plsc.load_gather(tab_v, [xv + _OFF[i]])
            wv = plsc.load_gather(w_v, [ids])
            acc = acc + wv * tv
        out_v[pl.ds(g * _L, _L)] = acc
    pltpu.sync_copy(out_v, out_hbm.at[pl.ds(wid * _ROWS, _ROWS)])


def kernel(X, weight, emb0, emb1, emb2, emb3, emb4, emb5, emb6, emb7, emb8,
           emb9, emb10, emb11, bias):
    tables = [emb0, emb1, emb2, emb3, emb4, emb5, emb6, emb7, emb8, emb9,
              emb10, emb11]
    flat_x = X.reshape(-1)
    flat_w = weight.reshape(-1)
    tab = jnp.concatenate([t.reshape(-1) for t in tables]
                          + [jnp.zeros((2,), jnp.float32)])
    bias16 = jnp.broadcast_to(bias.astype(jnp.float32), (_L,))
    return _sc_fm(flat_x, flat_w, tab, bias16)


# speedup vs baseline: 1.1936x; 1.1936x over previous
import functools

import jax
import jax.numpy as jnp
from jax import lax
from jax.experimental import pallas as pl
from jax.experimental.pallas import tpu as pltpu
from jax.experimental.pallas import tpu_sc as plsc

_SIZES = (6, 10, 2, 1, 1, 1, 1, 2, 1, 1, 2, 2)
_NF = 12
_B = 16384
_NC, _NS, _L = 2, 16, 16
_NW = _NC * _NS
_ROWS = _B // _NW
_CHUNK = _ROWS * _NF
_GROUPS = _ROWS // _L

_mesh = plsc.VectorSubcoreMesh(core_axis_name="c", subcore_axis_name="s")


@functools.partial(
    pl.kernel,
    out_type=jax.ShapeDtypeStruct((_B,), jnp.float32),
    mesh=_mesh,
    scratch_types=[
        pltpu.VMEM((_CHUNK,), jnp.int32),
        pltpu.VMEM((_CHUNK,), jnp.float32),
        pltpu.VMEM((13 * _L,), jnp.float32),
        pltpu.VMEM((_ROWS,), jnp.float32),
        pltpu.SemaphoreType.DMA,
    ],
    compiler_params=pltpu.CompilerParams(needs_layout_passes=False),
)
def _sc_fm(x_hbm, w_hbm, e0, e1, e2, e3, e4, e5, e6, e7, e8, e9, e10, e11,
           bias_hbm, out_hbm, x_v, w_v, tab_v, out_v, sem):
    wid = lax.axis_index("s") * _NC + lax.axis_index("c")
    base = wid * _CHUNK
    tabs = (e0, e1, e2, e3, e4, e5, e6, e7, e8, e9, e10, e11)
    copies = [
        pltpu.async_copy(x_hbm.at[pl.ds(base, _CHUNK)], x_v, sem),
        pltpu.async_copy(w_hbm.at[pl.ds(base, _CHUNK)], w_v, sem),
        pltpu.async_copy(bias_hbm, tab_v.at[pl.ds(12 * _L, 1)], sem),
    ]
    for i in range(_NF):
        copies.append(
            pltpu.async_copy(tabs[i], tab_v.at[pl.ds(i * _L, _SIZES[i])], sem)
        )
    for c in copies:
        c.wait()

    iota12 = lax.iota(jnp.int32, _L) * _NF
    bias_vec = plsc.load_gather(tab_v, [jnp.full((_L,), 12 * _L, jnp.int32)])

    @pl.loop(0, _GROUPS)
    def _group(g):
        ids = iota12 + g * (_L * _NF)
        acc = bias_vec
        for i in range(_NF):
            xv = plsc.load_gather(x_v, [ids + i])
            tv = plsc.load_gather(tab_v, [xv + i * _L])
            wv = plsc.load_gather(w_v, [ids + i])
            acc = acc + wv * tv
        out_v[pl.ds(g * _L, _L)] = acc

    pltpu.sync_copy(out_v, out_hbm.at[pl.ds(wid * _ROWS, _ROWS)])


def kernel(X, weight, emb0, emb1, emb2, emb3, emb4, emb5, emb6, emb7, emb8,
           emb9, emb10, emb11, bias):
    tabs = [t.reshape(-1) for t in
            (emb0, emb1, emb2, emb3, emb4, emb5, emb6, emb7, emb8, emb9,
             emb10, emb11)]
    return _sc_fm(X.reshape(-1), weight.reshape(-1), *tabs, bias)


# speedup vs baseline: 2.6317x; 2.2049x over previous
import functools

import jax
import jax.numpy as jnp
from jax import lax
from jax.experimental import pallas as pl
from jax.experimental.pallas import tpu as pltpu
from jax.experimental.pallas import tpu_sc as plsc

_SIZES = (6, 10, 2, 1, 1, 1, 1, 2, 1, 1, 2, 2)
_NF = 12
_B = 16384
_NC, _NS, _L = 2, 16, 16
_NW = _NC * _NS
_COLS = _B // _NW
_GROUPS = _COLS // _L

_mesh = plsc.VectorSubcoreMesh(core_axis_name="c", subcore_axis_name="s")


@functools.partial(
    pl.kernel,
    out_type=jax.ShapeDtypeStruct((_B,), jnp.float32),
    mesh=_mesh,
    scratch_types=[
        pltpu.VMEM((_NF, _COLS), jnp.int32),
        pltpu.VMEM((_NF, _COLS), jnp.float32),
        pltpu.VMEM((13 * _L,), jnp.float32),
        pltpu.VMEM((_COLS,), jnp.float32),
        pltpu.SemaphoreType.DMA,
    ],
    compiler_params=pltpu.CompilerParams(
        needs_layout_passes=False, use_tc_tiling_on_sc=True),
)
def _sc_fm(xt_hbm, wt_hbm, e0, e1, e2, e3, e4, e5, e6, e7, e8, e9, e10, e11,
           bias_hbm, out_hbm, x_v, w_v, tab_v, out_v, sem):
    wid = lax.axis_index("s") * _NC + lax.axis_index("c")
    col0 = wid * _COLS
    tabs = (e0, e1, e2, e3, e4, e5, e6, e7, e8, e9, e10, e11)
    copies = [
        pltpu.async_copy(xt_hbm.at[:, pl.ds(col0, _COLS)], x_v, sem),
        pltpu.async_copy(wt_hbm.at[:, pl.ds(col0, _COLS)], w_v, sem),
        pltpu.async_copy(bias_hbm, tab_v.at[pl.ds(12 * _L, 1)], sem),
    ]
    for i in range(_NF):
        copies.append(
            pltpu.async_copy(tabs[i], tab_v.at[pl.ds(i * _L, _SIZES[i])], sem)
        )
    for c in copies:
        c.wait()

    bias_vec = plsc.load_gather(tab_v, [jnp.full((_L,), 12 * _L, jnp.int32)])

    @pl.loop(0, _GROUPS)
    def _group(g):
        acc = bias_vec
        for i in range(_NF):
            xi = x_v[i, pl.ds(g * _L, _L)]
            wi = w_v[i, pl.ds(g * _L, _L)]
            tv = plsc.load_gather(tab_v, [xi + i * _L])
            acc = acc + wi * tv
        out_v[pl.ds(g * _L, _L)] = acc

    pltpu.sync_copy(out_v, out_hbm.at[pl.ds(col0, _COLS)])


def kernel(X, weight, emb0, emb1, emb2, emb3, emb4, emb5, emb6, emb7, emb8,
           emb9, emb10, emb11, bias):
    tabs = [t.reshape(-1) for t in
            (emb0, emb1, emb2, emb3, emb4, emb5, emb6, emb7, emb8, emb9,
             emb10, emb11)]
    return _sc_fm(X.T, weight.T, *tabs, bias)
